# Initial kernel scaffold; baseline (speedup 1.0000x reference)
import jax
import jax.numpy as jnp
from jax.experimental import pallas as pl


def kernel(x, w1_mat, w2_shift, w3_mat, b2, g1, be1, g2, be2, g3, be3):
    raise NotImplementedError("write your pallas kernel here")


# speedup vs baseline: 1.2880x; 1.2880x over previous
import functools

import jax
import jax.numpy as jnp
from jax import lax
from jax.experimental import pallas as pl
from jax.experimental.pallas import tpu as pltpu

_EPS = 1e-3
_HI = lax.Precision.HIGHEST


def _xstats_kernel(x_ref, sx_ref, m_ref):
    x = x_ref[0]
    sx_ref[0] = jnp.sum(x, axis=1, keepdims=True)
    xb = x.astype(jnp.bfloat16)
    m_ref[0] = lax.dot_general(xb, xb, (((1,), (1,)), ((), ())),
                               preferred_element_type=jnp.float32)


def _coef1_kernel(sxp_ref, mp_ref, w1_ref, g1_ref, be1_ref, a1_ref, c1_ref,
                  *, inv_m):
    sx = jnp.sum(sxp_ref[...], axis=0)
    m = jnp.sum(mp_ref[...], axis=0)
    w1 = w1_ref[...]
    s1 = jnp.dot(w1, sx, precision=_HI, preferred_element_type=jnp.float32)
    a = jnp.dot(w1, m, precision=_HI, preferred_element_type=jnp.float32)
    q1 = jnp.sum(a * w1, axis=1, keepdims=True)
    mean = s1 * inv_m
    var = q1 * inv_m - mean * mean
    a1 = g1_ref[...] * lax.rsqrt(var + _EPS)
    a1_ref[...] = a1
    c1_ref[...] = be1_ref[...] - mean * a1


def _fused2_kernel(x_ref, a1_ref, c1_ref, w1_ref, w2_ref, b2_ref,
                   y2_ref, s2_ref, q2_ref, *, H, W):
    hw = H * W
    y1 = jnp.dot(w1_ref[...], x_ref[0].astype(jnp.bfloat16),
                 preferred_element_type=jnp.float32)
    h = jnp.maximum(a1_ref[...] * y1 + c1_ref[...], 0.0)
    hb = h.astype(jnp.bfloat16)

    idx = lax.broadcasted_iota(jnp.int32, (1, hw), 1)
    row = idx // W
    col = idx % W

    acc = jnp.zeros((w2_ref.shape[1], hw), jnp.float32)
    for dy in (-1, 0, 1):
        for dx in (-1, 0, 1):
            s = dy * W + dx
            shifted = hb if s == 0 else pltpu.roll(hb, shift=(-s) % hw, axis=1)
            conds = []
            if dy == -1:
                conds.append(row >= 1)
            if dy == 1:
                conds.append(row <= H - 2)
            if dx == -1:
                conds.append(col >= 1)
            if dx == 1:
                conds.append(col <= W - 2)
            if conds:
                valid = functools.reduce(jnp.logical_and, conds)
                shifted = jnp.where(valid, shifted, 0)
            k = (dy + 1) * 3 + (dx + 1)
            acc = acc + jnp.dot(w2_ref[k], shifted,
                                preferred_element_type=jnp.float32)

    y2 = acc + b2_ref[...]
    y2_ref[0] = y2
    s2_ref[0] = jnp.sum(y2, axis=1, keepdims=True)
    q2_ref[0] = jnp.sum(y2 * y2, axis=1, keepdims=True)


def _fused3_kernel(y2_ref, s2p_ref, q2p_ref, g2_ref, be2_ref,
                   h2_ref, sh_ref, m2_ref, *, inv_m):
    mean = jnp.sum(s2p_ref[...], axis=0) * inv_m
    q = jnp.sum(q2p_ref[...], axis=0) * inv_m
    var = q - mean * mean
    a2 = g2_ref[...] * lax.rsqrt(var + _EPS)
    c2 = be2_ref[...] - mean * a2
    h = jnp.maximum(a2 * y2_ref[0] + c2, 0.0)
    hb = h.astype(jnp.bfloat16)
    h2_ref[0] = hb
    sh_ref[0] = jnp.sum(hb.astype(jnp.float32), axis=1, keepdims=True)
    m2_ref[0] = lax.dot_general(hb, hb, (((1,), (1,)), ((), ())),
                                preferred_element_type=jnp.float32)


def _coef3_kernel(shp_ref, m2p_ref, w3_ref, g3_ref, be3_ref, a3_ref, c3_ref,
                  *, inv_m):
    sh = jnp.sum(shp_ref[...], axis=0)
    m2 = jnp.sum(m2p_ref[...], axis=0)
    w3 = w3_ref[...]
    s3 = jnp.dot(w3, sh, precision=_HI, preferred_element_type=jnp.float32)
    a = jnp.dot(w3, m2, precision=_HI, preferred_element_type=jnp.float32)
    q3 = jnp.sum(a * w3, axis=1, keepdims=True)
    mean = s3 * inv_m
    var = q3 * inv_m - mean * mean
    a3 = g3_ref[...] * lax.rsqrt(var + _EPS)
    a3_ref[...] = a3
    c3_ref[...] = be3_ref[...] - mean * a3


def _out_kernel(h2_ref, w3_ref, a3_ref, c3_ref, o_ref):
    y3 = jnp.dot(w3_ref[...], h2_ref[0], preferred_element_type=jnp.float32)
    o_ref[0] = a3_ref[...] * y3 + c3_ref[...]


def kernel(x, w1_mat, w2_shift, w3_mat, b2, g1, be1, g2, be2, g3, be3):
    N, Cin, H, W = x.shape
    Cint = w1_mat.shape[0]
    Cout = w3_mat.shape[0]
    HW = H * W
    inv_m = 1.0 / float(N * HW)

    xr = x.reshape(N, Cin, HW)
    w1b = w1_mat.astype(jnp.bfloat16)
    w2b = w2_shift.astype(jnp.bfloat16)
    w3b = w3_mat.astype(jnp.bfloat16)

    col = lambda c: pl.BlockSpec((c, 1), lambda n: (0, 0))
    par = pltpu.CompilerParams(dimension_semantics=("parallel",))

    sxp, mp = pl.pallas_call(
        _xstats_kernel,
        out_shape=(jax.ShapeDtypeStruct((N, Cin, 1), jnp.float32),
                   jax.ShapeDtypeStruct((N, Cin, Cin), jnp.float32)),
        grid=(N,),
        in_specs=[pl.BlockSpec((1, Cin, HW), lambda n: (n, 0, 0))],
        out_specs=(pl.BlockSpec((1, Cin, 1), lambda n: (n, 0, 0)),
                   pl.BlockSpec((1, Cin, Cin), lambda n: (n, 0, 0))),
        compiler_params=par,
    )(xr)

    a1, c1 = pl.pallas_call(
        functools.partial(_coef1_kernel, inv_m=inv_m),
        out_shape=(jax.ShapeDtypeStruct((Cint, 1), jnp.float32),
                   jax.ShapeDtypeStruct((Cint, 1), jnp.float32)),
    )(sxp, mp, w1_mat, g1, be1)

    y2, s2p, q2p = pl.pallas_call(
        functools.partial(_fused2_kernel, H=H, W=W),
        out_shape=(jax.ShapeDtypeStruct((N, Cint, HW), jnp.float32),
                   jax.ShapeDtypeStruct((N, Cint, 1), jnp.float32),
                   jax.ShapeDtypeStruct((N, Cint, 1), jnp.float32)),
        grid=(N,),
        in_specs=[pl.BlockSpec((1, Cin, HW), lambda n: (n, 0, 0)),
                  col(Cint), col(Cint),
                  pl.BlockSpec((Cint, Cin), lambda n: (0, 0)),
                  pl.BlockSpec((9, Cint, Cint), lambda n: (0, 0, 0)),
                  col(Cint)],
        out_specs=(pl.BlockSpec((1, Cint, HW), lambda n: (n, 0, 0)),
                   pl.BlockSpec((1, Cint, 1), lambda n: (n, 0, 0)),
                   pl.BlockSpec((1, Cint, 1), lambda n: (n, 0, 0))),
        compiler_params=par,
    )(xr, a1, c1, w1b, w2b, b2)

    h2, shp, m2p = pl.pallas_call(
        functools.partial(_fused3_kernel, inv_m=inv_m),
        out_shape=(jax.ShapeDtypeStruct((N, Cint, HW), jnp.bfloat16),
                   jax.ShapeDtypeStruct((N, Cint, 1), jnp.float32),
                   jax.ShapeDtypeStruct((N, Cint, Cint), jnp.float32)),
        grid=(N,),
        in_specs=[pl.BlockSpec((1, Cint, HW), lambda n: (n, 0, 0)),
                  pl.BlockSpec((N, Cint, 1), lambda n: (0, 0, 0)),
                  pl.BlockSpec((N, Cint, 1), lambda n: (0, 0, 0)),
                  col(Cint), col(Cint)],
        out_specs=(pl.BlockSpec((1, Cint, HW), lambda n: (n, 0, 0)),
                   pl.BlockSpec((1, Cint, 1), lambda n: (n, 0, 0)),
                   pl.BlockSpec((1, Cint, Cint), lambda n: (n, 0, 0))),
        compiler_params=par,
    )(y2, s2p, q2p, g2, be2)

    a3, c3 = pl.pallas_call(
        functools.partial(_coef3_kernel, inv_m=inv_m),
        out_shape=(jax.ShapeDtypeStruct((Cout, 1), jnp.float32),
                   jax.ShapeDtypeStruct((Cout, 1), jnp.float32)),
    )(shp, m2p, w3_mat, g3, be3)

    out = pl.pallas_call(
        _out_kernel,
        out_shape=jax.ShapeDtypeStruct((N, Cout, HW), jnp.float32),
        grid=(N,),
        in_specs=[pl.BlockSpec((1, Cint, HW), lambda n: (n, 0, 0)),
                  pl.BlockSpec((Cout, Cint), lambda n: (0, 0)),
                  col(Cout), col(Cout)],
        out_specs=pl.BlockSpec((1, Cout, HW), lambda n: (n, 0, 0)),
        compiler_params=par,
    )(h2, w3b, a3, c3)

    return out.reshape(N, Cout, H, W)


# speedup vs baseline: 1.3874x; 1.0772x over previous
import functools

import jax
import jax.numpy as jnp
from jax import lax
from jax.experimental import pallas as pl
from jax.experimental.pallas import tpu as pltpu

_EPS = 1e-3
_HI = lax.Precision.HIGHEST


def _coeffs(g, be, s, q, inv_m):
    mean = s * inv_m
    var = q * inv_m - mean * mean
    a = g * lax.rsqrt(var + _EPS)
    return a, be - mean * a


def _conv3x3(hb, w2b, H, W):
    hw = H * W
    idx = lax.broadcasted_iota(jnp.int32, (1, hw), 1)
    row = idx // W
    col = idx % W
    acc = jnp.zeros((w2b.shape[1], hw), jnp.float32)
    for dy in (-1, 0, 1):
        for dx in (-1, 0, 1):
            s = dy * W + dx
            shifted = hb if s == 0 else pltpu.roll(hb, shift=(-s) % hw, axis=1)
            conds = []
            if dy == -1:
                conds.append(row >= 1)
            if dy == 1:
                conds.append(row <= H - 2)
            if dx == -1:
                conds.append(col >= 1)
            if dx == 1:
                conds.append(col <= W - 2)
            if conds:
                valid = functools.reduce(jnp.logical_and, conds)
                shifted = jnp.where(valid, shifted, 0)
            k = (dy + 1) * 3 + (dx + 1)
            acc = acc + jnp.dot(w2b[k], shifted,
                                preferred_element_type=jnp.float32)
    return acc


def _stage12_kernel(x_ref, w1_ref, w2_ref, b2_ref, g1_ref, be1_ref,
                    y2_ref, s2_ref, q2_ref,
                    xb_ref, m_ref, sx_ref, a1_ref, c1_ref, *, N, H, W):
    i = pl.program_id(0)

    @pl.when(i < N)
    def _phase_a():
        x = x_ref[0]
        xb = x.astype(jnp.bfloat16)
        xb_ref[i] = xb

        @pl.when(i == 0)
        def _():
            m_ref[...] = jnp.zeros_like(m_ref)
            sx_ref[...] = jnp.zeros_like(sx_ref)

        m_ref[...] += lax.dot_general(xb, xb, (((1,), (1,)), ((), ())),
                                      preferred_element_type=jnp.float32)
        sx_ref[...] += jnp.sum(x, axis=1, keepdims=True)

    @pl.when(i == N)
    def _coef1():
        w1 = w1_ref[...]
        s1 = jnp.dot(w1, sx_ref[...], precision=_HI,
                     preferred_element_type=jnp.float32)
        a = jnp.dot(w1, m_ref[...], precision=_HI,
                    preferred_element_type=jnp.float32)
        q1 = jnp.sum(a * w1, axis=1, keepdims=True)
        a1, c1 = _coeffs(g1_ref[...], be1_ref[...], s1, q1, 1.0 / (N * H * W))
        a1_ref[...] = a1
        c1_ref[...] = c1

    @pl.when(i >= N)
    def _phase_b():
        j = i - N
        w1b = w1_ref[...].astype(jnp.bfloat16)
        y1 = jnp.dot(w1b, xb_ref[j], preferred_element_type=jnp.float32)
        h = jnp.maximum(a1_ref[...] * y1 + c1_ref[...], 0.0)
        w2b = w2_ref[...].astype(jnp.bfloat16)
        y2 = _conv3x3(h.astype(jnp.bfloat16), w2b, H, W) + b2_ref[...]
        y2_ref[0] = y2.astype(jnp.bfloat16)

        @pl.when(i == N)
        def _():
            s2_ref[...] = jnp.zeros_like(s2_ref)
            q2_ref[...] = jnp.zeros_like(q2_ref)

        s2_ref[...] += jnp.sum(y2, axis=1, keepdims=True)
        q2_ref[...] += jnp.sum(y2 * y2, axis=1, keepdims=True)


def _stage34_kernel(y2_ref, s2_ref, q2_ref, g2_ref, be2_ref,
                    w3_ref, g3_ref, be3_ref,
                    o_ref,
                    hb_ref, m2_ref, sh_ref, a2_ref, c2_ref, a3_ref, c3_ref,
                    *, N, H, W):
    i = pl.program_id(0)
    inv_m = 1.0 / (N * H * W)

    @pl.when(i == 0)
    def _coef2():
        a2, c2 = _coeffs(g2_ref[...], be2_ref[...], s2_ref[...], q2_ref[...],
                         inv_m)
        a2_ref[...] = a2
        c2_ref[...] = c2
        m2_ref[...] = jnp.zeros_like(m2_ref)
        sh_ref[...] = jnp.zeros_like(sh_ref)

    @pl.when(i < N)
    def _phase_a():
        y2 = y2_ref[0].astype(jnp.float32)
        h = jnp.maximum(a2_ref[...] * y2 + c2_ref[...], 0.0)
        hb = h.astype(jnp.bfloat16)
        hb_ref[i] = hb
        m2_ref[...] += lax.dot_general(hb, hb, (((1,), (1,)), ((), ())),
                                       preferred_element_type=jnp.float32)
        sh_ref[...] += jnp.sum(hb.astype(jnp.float32), axis=1, keepdims=True)

    @pl.when(i == N)
    def _coef3():
        w3 = w3_ref[...]
        s3 = jnp.dot(w3, sh_ref[...], precision=_HI,
                     preferred_element_type=jnp.float32)
        a = jnp.dot(w3, m2_ref[...], precision=_HI,
                    preferred_element_type=jnp.float32)
        q3 = jnp.sum(a * w3, axis=1, keepdims=True)
        a3, c3 = _coeffs(g3_ref[...], be3_ref[...], s3, q3, inv_m)
        a3_ref[...] = a3
        c3_ref[...] = c3

    @pl.when(i >= N)
    def _phase_b():
        j = i - N
        w3b = w3_ref[...].astype(jnp.bfloat16)
        y3 = jnp.dot(w3b, hb_ref[j], preferred_element_type=jnp.float32)
        o_ref[0] = a3_ref[...] * y3 + c3_ref[...]


def kernel(x, w1_mat, w2_shift, w3_mat, b2, g1, be1, g2, be2, g3, be3):
    N, Cin, H, W = x.shape
    Cint = w1_mat.shape[0]
    Cout = w3_mat.shape[0]
    HW = H * W

    xr = x.reshape(N, Cin, HW)
    col = lambda c: pl.BlockSpec((c, 1), lambda i: (0, 0))
    arb = pltpu.CompilerParams(dimension_semantics=("arbitrary",))

    y2b, s2, q2 = pl.pallas_call(
        functools.partial(_stage12_kernel, N=N, H=H, W=W),
        out_shape=(jax.ShapeDtypeStruct((N, Cint, HW), jnp.bfloat16),
                   jax.ShapeDtypeStruct((Cint, 1), jnp.float32),
                   jax.ShapeDtypeStruct((Cint, 1), jnp.float32)),
        grid=(2 * N,),
        in_specs=[pl.BlockSpec((1, Cin, HW),
                               lambda i: (jnp.where(i < N, i, 0), 0, 0)),
                  pl.BlockSpec((Cint, Cin), lambda i: (0, 0)),
                  pl.BlockSpec((9, Cint, Cint), lambda i: (0, 0, 0)),
                  col(Cint), col(Cint), col(Cint)],
        out_specs=(pl.BlockSpec((1, Cint, HW),
                                lambda i: (jnp.where(i < N, 0, i - N), 0, 0)),
                   col(Cint), col(Cint)),
        scratch_shapes=[pltpu.VMEM((N, Cin, HW), jnp.bfloat16),
                        pltpu.VMEM((Cin, Cin), jnp.float32),
                        pltpu.VMEM((Cin, 1), jnp.float32),
                        pltpu.VMEM((Cint, 1), jnp.float32),
                        pltpu.VMEM((Cint, 1), jnp.float32)],
        compiler_params=arb,
    )(xr, w1_mat, w2_shift, b2, g1, be1)

    out = pl.pallas_call(
        functools.partial(_stage34_kernel, N=N, H=H, W=W),
        out_shape=jax.ShapeDtypeStruct((N, Cout, HW), jnp.float32),
        grid=(2 * N,),
        in_specs=[pl.BlockSpec((1, Cint, HW),
                               lambda i: (jnp.where(i < N, i, 0), 0, 0)),
                  col(Cint), col(Cint), col(Cint), col(Cint),
                  pl.BlockSpec((Cout, Cint), lambda i: (0, 0)),
                  col(Cout), col(Cout)],
        out_specs=pl.BlockSpec((1, Cout, HW),
                               lambda i: (jnp.where(i < N, 0, i - N), 0, 0)),
        scratch_shapes=[pltpu.VMEM((N, Cint, HW), jnp.bfloat16),
                        pltpu.VMEM((Cint, Cint), jnp.float32),
                        pltpu.VMEM((Cint, 1), jnp.float32),
                        pltpu.VMEM((Cint, 1), jnp.float32),
                        pltpu.VMEM((Cint, 1), jnp.float32),
                        pltpu.VMEM((Cout, 1), jnp.float32),
                        pltpu.VMEM((Cout, 1), jnp.float32)],
        compiler_params=arb,
    )(y2b, s2, q2, g2, be2, w3_mat, g3, be3)

    return out.reshape(N, Cout, H, W)
